# baseline (speedup 1.0000x reference)
import functools

import jax
import jax.numpy as jnp
from jax import lax
from jax.experimental import pallas as pl
from jax.experimental.pallas import tpu as pltpu
from jax.experimental.pallas import tpu_sc as plsc

N = 10000
E = 320000
D = 128

NC = 2
NS = 16
NW = NC * NS
EPW = E // NW

TBL = 1600
CH2 = 48
CL = 2016
CHI = 8832
NLCH = CL // CH2
NHCH = CHI // CH2
SB = 8
NSB = NHCH // SB
TRASH = N
NACC = N + 16
ZRT = NACC // NS
RPT = N // NS
TPT = TBL // NS

BR = 1000
GRID = N // BR
BR1 = 1024
GRID1 = 10

DEGP = 10240

_mesh = plsc.VectorSubcoreMesh(core_axis_name="c", subcore_axis_name="s")
_params = pltpu.CompilerParams(needs_layout_passes=False,
                               use_tc_tiling_on_sc=False)


@functools.partial(
    pl.kernel,
    out_type=[
        jax.ShapeDtypeStruct((NW, DEGP), jnp.float32),
        jax.ShapeDtypeStruct((NW, CL), jnp.int32),
        jax.ShapeDtypeStruct((NW, CL), jnp.int32),
        jax.ShapeDtypeStruct((NW, CHI), jnp.int32),
        jax.ShapeDtypeStruct((NW, CHI), jnp.int32),
    ],
    mesh=_mesh,
    compiler_params=_params,
    scratch_types=[
        pltpu.VMEM((DEGP,), jnp.float32),
        pltpu.VMEM((EPW,), jnp.int32),
        pltpu.VMEM((EPW,), jnp.int32),
        pltpu.VMEM((CL + 16,), jnp.int32),
        pltpu.VMEM((CL + 16,), jnp.int32),
        pltpu.VMEM((CHI + 16,), jnp.int32),
        pltpu.VMEM((CHI + 16,), jnp.int32),
    ],
)
def _prep_kernel(src_hbm, dst_hbm, deg_o, ls_o, ld_o, hs_o, hd_o,
                 dv, sv, dvv, lsv, ldv, hsv, hdv):
    cid = lax.axis_index("c")
    sid = lax.axis_index("s")
    wid = cid * NS + sid

    def zf(ref, n, val, dt):
        vec = jnp.full((16,), val, dt)

        def body(i, carry):
            ref[pl.ds(i * 16, 16)] = vec
            return carry

        lax.fori_loop(0, n // 16, body, 0)

    zf(dv, DEGP, 0.0, jnp.float32)
    zf(lsv, CL + 16, 0, jnp.int32)
    zf(ldv, CL + 16, TRASH, jnp.int32)
    zf(hsv, CHI + 16, 0, jnp.int32)
    zf(hdv, CHI + 16, TRASH, jnp.int32)

    base = pl.multiple_of(wid * EPW, 8)
    pltpu.sync_copy(src_hbm.at[pl.ds(base, EPW)], sv)
    pltpu.sync_copy(dst_hbm.at[pl.ds(base, EPW)], dvv)

    ones = jnp.ones((16,), jnp.float32)

    def body(j, carry):
        cur_l, cur_h = carry
        s16 = sv[pl.ds(j * 16, 16)]
        d16 = dvv[pl.ds(j * 16, 16)]
        plsc.addupdate_scatter(dv, [d16], ones)
        m = s16 < TBL
        nl = jnp.sum(m.astype(jnp.int32))
        cl = jnp.minimum(cur_l, CL)
        ch = jnp.minimum(cur_h, CHI)
        plsc.store_compressed(lsv.at[pl.ds(cl, 16)], s16, mask=m)
        plsc.store_compressed(ldv.at[pl.ds(cl, 16)], d16, mask=m)
        nm = jnp.logical_not(m)
        plsc.store_compressed(hsv.at[pl.ds(ch, 16)], s16, mask=nm)
        plsc.store_compressed(hdv.at[pl.ds(ch, 16)], d16, mask=nm)
        return cur_l + nl, cur_h + (16 - nl)

    lax.fori_loop(0, EPW // 16, body, (jnp.int32(0), jnp.int32(0)))

    pltpu.sync_copy(dv, deg_o.at[wid])
    pltpu.sync_copy(lsv.at[pl.ds(0, CL)], ls_o.at[wid])
    pltpu.sync_copy(ldv.at[pl.ds(0, CL)], ld_o.at[wid])
    pltpu.sync_copy(hsv.at[pl.ds(0, CHI)], hs_o.at[wid])
    pltpu.sync_copy(hdv.at[pl.ds(0, CHI)], hd_o.at[wid])


@functools.partial(
    pl.kernel,
    out_type=jax.ShapeDtypeStruct((NC, N, D), jnp.float32),
    mesh=_mesh,
    compiler_params=_params,
    scratch_types=[
        pltpu.VMEM_SHARED((NACC, D), jnp.float32),
        pltpu.VMEM_SHARED((TBL, D), jnp.float32),
        pltpu.VMEM((NLCH, CH2), jnp.int32),
        pltpu.VMEM((NLCH, CH2), jnp.int32),
        pltpu.VMEM((2, SB, CH2), jnp.int32),
        pltpu.VMEM((2, SB, CH2), jnp.int32),
        pltpu.VMEM((CH2, D), jnp.float32),
        pltpu.VMEM((CH2, D), jnp.float32),
        pltpu.VMEM((CH2, D), jnp.float32),
        pltpu.VMEM((CH2, D), jnp.float32),
        pltpu.VMEM((CH2, D), jnp.float32),
        pltpu.SemaphoreType.DMA,
        pltpu.SemaphoreType.DMA,
        pltpu.SemaphoreType.DMA,
        pltpu.SemaphoreType.DMA,
        pltpu.SemaphoreType.DMA,
        pltpu.SemaphoreType.DMA,
        pltpu.SemaphoreType.DMA,
        pltpu.SemaphoreType.DMA,
        pltpu.SemaphoreType.DMA,
        pltpu.SemaphoreType.DMA,
        pltpu.SemaphoreType.DMA,
        pltpu.SemaphoreType.DMA,
        pltpu.SemaphoreType.DMA,
    ],
)
def _scatter_kernel(g_hbm, ls_hbm, ld_hbm, hs_hbm, hd_hbm, out_hbm,
                    acc, tbl, lsb, ldb, hsb, hdb,
                    lrb0, lrb1, hrb0, hrb1, hrb2,
                    lg0, lg1, lu0, lu1, hg0, hg1, hg2, hu0, hu1, hu2,
                    st0, st1, six):
    cid = lax.axis_index("c")
    sid = lax.axis_index("s")
    wid = cid * NS + sid
    lrb = (lrb0, lrb1)
    lgs = (lg0, lg1)
    lus = (lu0, lu1)
    hrb = (hrb0, hrb1, hrb2)
    hgs = (hg0, hg1, hg2)
    hus = (hu0, hu1, hu2)
    sts = (st0, st1)

    ix0 = pltpu.async_copy(ls_hbm.at[wid], lsb, six)
    ix1 = pltpu.async_copy(ld_hbm.at[wid], ldb, six)

    def hstage_start(sb):
        b = sb % 2
        pltpu.async_copy(hs_hbm.at[wid, pl.ds(sb * SB, SB)], hsb.at[b],
                         sts[b])
        pltpu.async_copy(hd_hbm.at[wid, pl.ds(sb * SB, SB)], hdb.at[b],
                         sts[b])

    def hstage_wait(sb):
        b = sb % 2
        pltpu.make_async_copy(hs_hbm.at[wid, pl.ds(sb * SB, SB)], hsb.at[b],
                              sts[b]).wait()
        pltpu.make_async_copy(hd_hbm.at[wid, pl.ds(sb * SB, SB)], hdb.at[b],
                              sts[b]).wait()

    hstage_start(0)
    hstage_start(1)

    pltpu.sync_copy(g_hbm.at[pl.ds(sid * TPT, TPT)],
                    tbl.at[pl.ds(sid * TPT, TPT)])

    def zero_rows(i, carry):
        def zrow(j, inner):
            hrb0[i, pl.ds(j * 16, 16)] = jnp.zeros((16,), jnp.float32)
            return inner
        return lax.fori_loop(0, D // 16, zrow, carry)

    lax.fori_loop(0, CH2, zero_rows, 0)

    for k in range(ZRT // CH2):
        pltpu.sync_copy(hrb0, acc.at[pl.ds(sid * ZRT + k * CH2, CH2)])
    zrem = ZRT - (ZRT // CH2) * CH2
    if zrem:
        pltpu.sync_copy(hrb0.at[pl.ds(0, zrem)],
                        acc.at[pl.ds(sid * ZRT + (ZRT // CH2) * CH2, zrem)])
    ix0.wait()
    ix1.wait()
    plsc.subcore_barrier()

    def lg_start(k):
        pltpu.async_copy(tbl.at[lsb.at[k]], lrb[k % 2], lgs[k % 2])

    def lg_wait(k):
        pltpu.make_async_copy(tbl.at[lsb.at[k]], lrb[k % 2],
                              lgs[k % 2]).wait()

    def lsc_start(k):
        pltpu.async_copy(lrb[k % 2], acc.at[ldb.at[k]], lus[k % 2], add=True)

    def lsc_wait(k):
        pltpu.make_async_copy(lrb[k % 2], acc.at[ldb.at[k]],
                              lus[k % 2]).wait()

    def hg_start(h):
        b = (h // SB) % 2
        pltpu.async_copy(g_hbm.at[hsb.at[b, h % SB]], hrb[h % 3], hgs[h % 3])

    def hg_wait(h):
        b = (h // SB) % 2
        pltpu.make_async_copy(g_hbm.at[hsb.at[b, h % SB]], hrb[h % 3],
                              hgs[h % 3]).wait()

    def hsc_start(h):
        b = (h // SB) % 2
        pltpu.async_copy(hrb[h % 3], acc.at[hdb.at[b, h % SB]], hus[h % 3],
                         add=True)

    def hsc_wait(h):
        b = (h // SB) % 2
        pltpu.make_async_copy(hrb[h % 3], acc.at[hdb.at[b, h % SB]],
                              hus[h % 3]).wait()

    events = []
    hh = 0
    for r in range(NLCH):
        events.append(("L", r))
        for _ in range(NHCH // NLCH + (1 if r < NHCH % NLCH else 0)):
            events.append(("H", hh))
            hh += 1
    assert hh == NHCH

    hstage_wait(0)
    lg_start(0)
    hg_start(0)
    hg_start(1)
    staged = 1
    waited_sb = 0

    for kind, c in events:
        if kind == "L":
            if c >= 1:
                lsc_wait(c - 1)
            if c + 1 < NLCH:
                lg_start(c + 1)
            lg_wait(c)
            lsc_start(c)
        else:
            hg_wait(c)
            hsc_start(c)
            if c >= 1:
                hsc_wait(c - 1)
            if c % SB == 0:
                s_next = c // SB + 1
                if s_next < NSB and s_next > staged:
                    hstage_start(s_next)
                    staged = s_next
            nxt = c + 2
            if nxt < NHCH:
                sb = nxt // SB
                if sb > waited_sb:
                    hstage_wait(sb)
                    waited_sb = sb
                hg_start(nxt)

    lsc_wait(NLCH - 1)
    hsc_wait(NHCH - 1)

    plsc.subcore_barrier()
    pltpu.sync_copy(acc.at[pl.ds(sid * RPT, RPT)],
                    out_hbm.at[cid, pl.ds(sid * RPT, RPT), :])


def _in_body(x_ref, wi_ref, bi_ref, w0_ref, dp_ref, h_ref, g_ref, dc_ref):
    cd = (((1,), (1,)), ((), ()))
    h = lax.dot_general(x_ref[...], wi_ref[...], cd,
                        preferred_element_type=jnp.float32) + bi_ref[...]
    h_ref[...] = h
    hw = lax.dot_general(h, w0_ref[...], cd,
                         preferred_element_type=jnp.float32)
    ones = jnp.ones((NW, 1), jnp.float32)
    s = lax.dot_general(dp_ref[...], ones, (((0,), (0,)), ((), ())),
                        preferred_element_type=jnp.float32)
    dinv = lax.rsqrt(1.0 + s)
    dc_ref[...] = dinv
    g_ref[...] = hw * dinv


def _mid_body(h_ref, g_ref, pa_ref, pb_ref, dinv_ref, cb_ref, lg_ref, lb_ref,
              w1_ref, h1_ref, g1_ref):
    dinv = dinv_ref[...]
    y = dinv * (pa_ref[0] + pb_ref[0] + g_ref[...]) + cb_ref[...]
    m = jnp.mean(y, axis=1, keepdims=True)
    c = y - m
    v = jnp.mean(c * c, axis=1, keepdims=True)
    yn = c * lax.rsqrt(v + 1e-5) * lg_ref[...] + lb_ref[...]
    h1 = h_ref[...] + jnp.maximum(yn, 0.0)
    h1_ref[...] = h1
    cd = (((1,), (1,)), ((), ()))
    g1_ref[...] = lax.dot_general(h1, w1_ref[...], cd,
                                  preferred_element_type=jnp.float32) * dinv


def _out_body(h_ref, g_ref, pa_ref, pb_ref, dinv_ref, cb_ref, lg_ref, lb_ref,
              ow_ref, ob_ref, o_ref):
    dinv = dinv_ref[...]
    y = dinv * (pa_ref[0] + pb_ref[0] + g_ref[...]) + cb_ref[...]
    m = jnp.mean(y, axis=1, keepdims=True)
    c = y - m
    v = jnp.mean(c * c, axis=1, keepdims=True)
    yn = c * lax.rsqrt(v + 1e-5) * lg_ref[...] + lb_ref[...]
    h2 = h_ref[...] + jnp.maximum(yn, 0.0)
    cd = (((1,), (1,)), ((), ()))
    o_ref[...] = lax.dot_general(h2, ow_ref[...], cd,
                                 preferred_element_type=jnp.float32) + ob_ref[...]


def _row_spec():
    return pl.BlockSpec((BR, D), lambda i: (i, 0))


def _w_spec():
    return pl.BlockSpec((D, D), lambda i: (0, 0))


def _v_spec():
    return pl.BlockSpec((1, D), lambda i: (0, 0))


def _dinv_spec():
    return pl.BlockSpec((BR, 1), lambda i: (i, 0))


def _p_spec(c):
    return pl.BlockSpec((1, BR, D), lambda i, c=c: (c, i, 0))


_f32 = jnp.float32


def kernel(x, edge_index, in_w, in_b, conv_w0, conv_b0, ln_g0, ln_b0,
           conv_w1, conv_b1, ln_g1, ln_b1, out_w, out_b):
    src_flat = edge_index[0]
    dst_flat = edge_index[1]

    degp, ls, ld, hs, hd = _prep_kernel(src_flat, dst_flat)
    ls3 = ls.reshape(NW, NLCH, CH2)
    ld3 = ld.reshape(NW, NLCH, CH2)
    hs3 = hs.reshape(NW, NHCH, CH2)
    hd3 = hd.reshape(NW, NHCH, CH2)

    h, g0, dinv_col = pl.pallas_call(
        _in_body,
        grid=(GRID1,),
        in_specs=[pl.BlockSpec((BR1, D), lambda i: (i, 0)),
                  pl.BlockSpec((D, D), lambda i: (0, 0)),
                  pl.BlockSpec((1, D), lambda i: (0, 0)),
                  pl.BlockSpec((D, D), lambda i: (0, 0)),
                  pl.BlockSpec((NW, BR1), lambda i: (0, i))],
        out_specs=[pl.BlockSpec((BR1, D), lambda i: (i, 0)),
                   pl.BlockSpec((BR1, D), lambda i: (i, 0)),
                   pl.BlockSpec((BR1, 1), lambda i: (i, 0))],
        out_shape=[jax.ShapeDtypeStruct((N, D), _f32),
                   jax.ShapeDtypeStruct((N, D), _f32),
                   jax.ShapeDtypeStruct((N, 1), _f32)],
    )(x, in_w, in_b.reshape(1, D), conv_w0, degp)

    p0 = _scatter_kernel(g0, ls3, ld3, hs3, hd3)

    h1, g1 = pl.pallas_call(
        _mid_body,
        grid=(GRID,),
        in_specs=[_row_spec(), _row_spec(), _p_spec(0), _p_spec(1),
                  _dinv_spec(), _v_spec(), _v_spec(), _v_spec(), _w_spec()],
        out_specs=[_row_spec(), _row_spec()],
        out_shape=[jax.ShapeDtypeStruct((N, D), _f32),
                   jax.ShapeDtypeStruct((N, D), _f32)],
    )(h, g0, p0, p0, dinv_col, conv_b0.reshape(1, D), ln_g0.reshape(1, D),
      ln_b0.reshape(1, D), conv_w1)

    p1 = _scatter_kernel(g1, ls3, ld3, hs3, hd3)

    out = pl.pallas_call(
        _out_body,
        grid=(GRID,),
        in_specs=[_row_spec(), _row_spec(), _p_spec(0), _p_spec(1),
                  _dinv_spec(), _v_spec(), _v_spec(), _v_spec(), _w_spec(),
                  _v_spec()],
        out_specs=_row_spec(),
        out_shape=jax.ShapeDtypeStruct((N, D), _f32),
    )(h1, g1, p1, p1, dinv_col, conv_b1.reshape(1, D), ln_g1.reshape(1, D),
      ln_b1.reshape(1, D), out_w, out_b.reshape(1, D))

    return out


# speedup vs baseline: 4.3268x; 4.3268x over previous
import functools

import jax
import jax.numpy as jnp
from jax import lax
from jax.experimental import pallas as pl
from jax.experimental.pallas import tpu as pltpu
from jax.experimental.pallas import tpu_sc as plsc

N = 10000
E = 320000
D = 128

NC = 2
NS = 16
NW = NC * NS
EPW = E // NW
CH = 80
NCH = EPW // CH
RPT = N // NS
ZR = 125
BR = 1000
GRID = N // BR
BR1 = 1024
GRID1 = 10

DEGP = 10240

_mesh = plsc.VectorSubcoreMesh(core_axis_name="c", subcore_axis_name="s")


@functools.partial(
    pl.kernel,
    out_type=jax.ShapeDtypeStruct((NW, DEGP), jnp.float32),
    mesh=_mesh,
    compiler_params=pltpu.CompilerParams(needs_layout_passes=False, use_tc_tiling_on_sc=False),
    scratch_types=[
        pltpu.VMEM((DEGP,), jnp.float32),
        pltpu.VMEM((EPW,), jnp.int32),
    ],
)
def _deg_kernel(dst_hbm, out_hbm, dv, dbuf):
    cid = lax.axis_index("c")
    sid = lax.axis_index("s")
    wid = cid * NS + sid

    def zero_body(i, carry):
        dv[pl.ds(i * 16, 16)] = jnp.zeros((16,), jnp.float32)
        return carry

    lax.fori_loop(0, DEGP // 16, zero_body, 0)

    base = pl.multiple_of(wid * EPW, 8)
    pltpu.sync_copy(dst_hbm.at[pl.ds(base, EPW)], dbuf)

    ones = jnp.ones((16,), jnp.float32)

    def acc_body(j, carry):
        idx = dbuf[pl.ds(j * 16, 16)]
        plsc.addupdate_scatter(dv, [idx], ones)
        return carry

    lax.fori_loop(0, EPW // 16, acc_body, 0)
    pltpu.sync_copy(dv, out_hbm.at[wid])


@functools.partial(
    pl.kernel,
    out_type=jax.ShapeDtypeStruct((NC, N, D), jnp.float32),
    mesh=_mesh,
    compiler_params=pltpu.CompilerParams(needs_layout_passes=False, use_tc_tiling_on_sc=False),
    scratch_types=[
        pltpu.VMEM_SHARED((N, D), jnp.float32),
        pltpu.VMEM((NCH, CH), jnp.int32),
        pltpu.VMEM((NCH, CH), jnp.int32),
        pltpu.VMEM((CH, D), jnp.float32),
        pltpu.VMEM((CH, D), jnp.float32),
        pltpu.VMEM((CH, D), jnp.float32),
        pltpu.SemaphoreType.DMA,
        pltpu.SemaphoreType.DMA,
        pltpu.SemaphoreType.DMA,
        pltpu.SemaphoreType.DMA,
        pltpu.SemaphoreType.DMA,
        pltpu.SemaphoreType.DMA,
        pltpu.SemaphoreType.DMA,
    ],
)
def _scatter_kernel(g_hbm, src_hbm, dst_hbm, out_hbm,
                    acc, sbuf, dbuf, rb0, rb1, rb2,
                    sg0, sg1, sg2, ss0, ss1, ss2, six):
    cid = lax.axis_index("c")
    sid = lax.axis_index("s")
    wid = cid * NS + sid
    rbs = (rb0, rb1, rb2)
    sgs = (sg0, sg1, sg2)
    sss = (ss0, ss1, ss2)

    ix_s = pltpu.async_copy(src_hbm.at[pl.ds(wid * NCH, NCH)], sbuf, six)
    ix_d = pltpu.async_copy(dst_hbm.at[pl.ds(wid * NCH, NCH)], dbuf, six)

    def zero_rows(i, carry):
        def zrow(j, inner):
            rb0[i, pl.ds(j * 16, 16)] = jnp.zeros((16,), jnp.float32)
            return inner
        return lax.fori_loop(0, D // 16, zrow, carry)

    lax.fori_loop(0, CH, zero_rows, 0)

    for k in range(RPT // CH):
        pltpu.sync_copy(rb0, acc.at[pl.ds(sid * RPT + k * CH, CH)])
    rem = RPT - (RPT // CH) * CH
    if rem:
        pltpu.sync_copy(rb0.at[pl.ds(0, rem)],
                        acc.at[pl.ds(sid * RPT + (RPT // CH) * CH, rem)])
    ix_s.wait()
    ix_d.wait()
    plsc.subcore_barrier()

    def g_start(k, b):
        pltpu.async_copy(g_hbm.at[sbuf.at[k]], rbs[b], sgs[b])

    def g_wait(k, b):
        pltpu.make_async_copy(g_hbm.at[sbuf.at[k]], rbs[b], sgs[b]).wait()

    def s_start(k, b):
        pltpu.async_copy(rbs[b], acc.at[dbuf.at[k]], sss[b], add=True)

    def s_wait(k, b):
        pltpu.make_async_copy(rbs[b], acc.at[dbuf.at[k]], sss[b]).wait()

    for b in range(3):
        g_start(b, b)

    NP = (NCH - 5) // 3

    def ring_body(p, carry):
        k = 3 * p
        for b in range(3):
            g_wait(k + b, b)
            s_start(k + b, b)
        for b in range(3):
            s_wait(k + b, b)
            g_start(k + 3 + b, b)
        return carry

    lax.fori_loop(0, NP, ring_body, 0)

    base = 3 * NP
    for b in range(3):
        g_wait(base + b, b)
        s_start(base + b, b)
    for b in range(2):
        s_wait(base + b, b)
        g_start(base + 3 + b, b)
    s_wait(base + 2, 2)
    for b in range(2):
        g_wait(base + 3 + b, b)
        s_start(base + 3 + b, b)
        s_wait(base + 3 + b, b)

    plsc.subcore_barrier()
    pltpu.sync_copy(acc.at[pl.ds(sid * RPT, RPT)],
                    out_hbm.at[cid, pl.ds(sid * RPT, RPT), :])


def _in_body(x_ref, wi_ref, bi_ref, w0_ref, dp_ref, h_ref, g_ref, dc_ref):
    cd = (((1,), (1,)), ((), ()))
    h = lax.dot_general(x_ref[...], wi_ref[...], cd,
                        preferred_element_type=jnp.float32) + bi_ref[...]
    h_ref[...] = h
    hw = lax.dot_general(h, w0_ref[...], cd,
                         preferred_element_type=jnp.float32)
    ones = jnp.ones((NW, 1), jnp.float32)
    s = lax.dot_general(dp_ref[...], ones, (((0,), (0,)), ((), ())),
                        preferred_element_type=jnp.float32)
    dinv = lax.rsqrt(1.0 + s)
    dc_ref[...] = dinv
    g_ref[...] = hw * dinv


def _mid_body(h_ref, g_ref, pa_ref, pb_ref, dinv_ref, cb_ref, lg_ref, lb_ref,
              w1_ref, h1_ref, g1_ref):
    dinv = dinv_ref[...]
    y = dinv * (pa_ref[0] + pb_ref[0] + g_ref[...]) + cb_ref[...]
    m = jnp.mean(y, axis=1, keepdims=True)
    c = y - m
    v = jnp.mean(c * c, axis=1, keepdims=True)
    yn = c * lax.rsqrt(v + 1e-5) * lg_ref[...] + lb_ref[...]
    h1 = h_ref[...] + jnp.maximum(yn, 0.0)
    h1_ref[...] = h1
    cd = (((1,), (1,)), ((), ()))
    g1_ref[...] = lax.dot_general(h1, w1_ref[...], cd,
                                  preferred_element_type=jnp.float32) * dinv


def _out_body(h_ref, g_ref, pa_ref, pb_ref, dinv_ref, cb_ref, lg_ref, lb_ref,
              ow_ref, ob_ref, o_ref):
    dinv = dinv_ref[...]
    y = dinv * (pa_ref[0] + pb_ref[0] + g_ref[...]) + cb_ref[...]
    m = jnp.mean(y, axis=1, keepdims=True)
    c = y - m
    v = jnp.mean(c * c, axis=1, keepdims=True)
    yn = c * lax.rsqrt(v + 1e-5) * lg_ref[...] + lb_ref[...]
    h2 = h_ref[...] + jnp.maximum(yn, 0.0)
    cd = (((1,), (1,)), ((), ()))
    o_ref[...] = lax.dot_general(h2, ow_ref[...], cd,
                                 preferred_element_type=jnp.float32) + ob_ref[...]


def _row_spec():
    return pl.BlockSpec((BR, D), lambda i: (i, 0))


def _w_spec():
    return pl.BlockSpec((D, D), lambda i: (0, 0))


def _v_spec():
    return pl.BlockSpec((1, D), lambda i: (0, 0))


def _dinv_spec():
    return pl.BlockSpec((BR, 1), lambda i: (i, 0))


def _p_spec(c):
    return pl.BlockSpec((1, BR, D), lambda i, c=c: (c, i, 0))


_f32 = jnp.float32


def kernel(x, edge_index, in_w, in_b, conv_w0, conv_b0, ln_g0, ln_b0,
           conv_w1, conv_b1, ln_g1, ln_b1, out_w, out_b):
    src2d = edge_index[0].reshape(E // CH, CH)
    dst2d = edge_index[1].reshape(E // CH, CH)
    dst_flat = edge_index[1]

    degp = _deg_kernel(dst_flat)

    h, g0, dinv_col = pl.pallas_call(
        _in_body,
        grid=(GRID1,),
        in_specs=[pl.BlockSpec((BR1, D), lambda i: (i, 0)),
                  pl.BlockSpec((D, D), lambda i: (0, 0)),
                  pl.BlockSpec((1, D), lambda i: (0, 0)),
                  pl.BlockSpec((D, D), lambda i: (0, 0)),
                  pl.BlockSpec((NW, BR1), lambda i: (0, i))],
        out_specs=[pl.BlockSpec((BR1, D), lambda i: (i, 0)),
                   pl.BlockSpec((BR1, D), lambda i: (i, 0)),
                   pl.BlockSpec((BR1, 1), lambda i: (i, 0))],
        out_shape=[jax.ShapeDtypeStruct((N, D), _f32),
                   jax.ShapeDtypeStruct((N, D), _f32),
                   jax.ShapeDtypeStruct((N, 1), _f32)],
    )(x, in_w, in_b.reshape(1, D), conv_w0, degp)

    p0 = _scatter_kernel(g0, src2d, dst2d)

    h1, g1 = pl.pallas_call(
        _mid_body,
        grid=(GRID,),
        in_specs=[_row_spec(), _row_spec(), _p_spec(0), _p_spec(1),
                  _dinv_spec(), _v_spec(), _v_spec(), _v_spec(), _w_spec()],
        out_specs=[_row_spec(), _row_spec()],
        out_shape=[jax.ShapeDtypeStruct((N, D), _f32),
                   jax.ShapeDtypeStruct((N, D), _f32)],
    )(h, g0, p0, p0, dinv_col, conv_b0.reshape(1, D), ln_g0.reshape(1, D),
      ln_b0.reshape(1, D), conv_w1)

    p1 = _scatter_kernel(g1, src2d, dst2d)

    out = pl.pallas_call(
        _out_body,
        grid=(GRID,),
        in_specs=[_row_spec(), _row_spec(), _p_spec(0), _p_spec(1),
                  _dinv_spec(), _v_spec(), _v_spec(), _v_spec(), _w_spec(),
                  _v_spec()],
        out_specs=_row_spec(),
        out_shape=jax.ShapeDtypeStruct((N, D), _f32),
    )(h1, g1, p1, p1, dinv_col, conv_b1.reshape(1, D), ln_g1.reshape(1, D),
      ln_b1.reshape(1, D), out_w, out_b.reshape(1, D))

    return out


# speedup vs baseline: 4.4032x; 1.0176x over previous
import functools

import jax
import jax.numpy as jnp
from jax import lax
from jax.experimental import pallas as pl
from jax.experimental.pallas import tpu as pltpu
from jax.experimental.pallas import tpu_sc as plsc

N = 10000
E = 320000
D = 128

NC = 2
NS = 16
NW = NC * NS
EPW = E // NW
CH = 80
NCH = EPW // CH
RPT = N // NS
ZR = 125
BR = 1000
GRID = N // BR
BR1 = 1024
GRID1 = 10

DEGP = 10240

_mesh = plsc.VectorSubcoreMesh(core_axis_name="c", subcore_axis_name="s")


@functools.partial(
    pl.kernel,
    out_type=jax.ShapeDtypeStruct((NW, DEGP), jnp.float32),
    mesh=_mesh,
    compiler_params=pltpu.CompilerParams(needs_layout_passes=False, use_tc_tiling_on_sc=False),
    scratch_types=[
        pltpu.VMEM((DEGP,), jnp.float32),
        pltpu.VMEM((EPW,), jnp.int32),
    ],
)
def _deg_kernel(dst_hbm, out_hbm, dv, dbuf):
    cid = lax.axis_index("c")
    sid = lax.axis_index("s")
    wid = cid * NS + sid

    def zero_body(i, carry):
        dv[pl.ds(i * 16, 16)] = jnp.zeros((16,), jnp.float32)
        return carry

    lax.fori_loop(0, DEGP // 16, zero_body, 0)

    base = pl.multiple_of(wid * EPW, 8)
    pltpu.sync_copy(dst_hbm.at[pl.ds(base, EPW)], dbuf)

    ones = jnp.ones((16,), jnp.float32)

    def acc_body(j, carry):
        idx = dbuf[pl.ds(j * 16, 16)]
        plsc.addupdate_scatter(dv, [idx], ones)
        return carry

    lax.fori_loop(0, EPW // 16, acc_body, 0)
    pltpu.sync_copy(dv, out_hbm.at[wid])


@functools.partial(
    pl.kernel,
    out_type=jax.ShapeDtypeStruct((NC, N, D), jnp.float32),
    mesh=_mesh,
    compiler_params=pltpu.CompilerParams(needs_layout_passes=False, use_tc_tiling_on_sc=False),
    scratch_types=[
        pltpu.VMEM_SHARED((N, D), jnp.float32),
        pltpu.VMEM((NCH, CH), jnp.int32),
        pltpu.VMEM((NCH, CH), jnp.int32),
        pltpu.VMEM((CH, D), jnp.float32),
        pltpu.VMEM((CH, D), jnp.float32),
        pltpu.SemaphoreType.DMA,
        pltpu.SemaphoreType.DMA,
        pltpu.SemaphoreType.DMA,
    ],
)
def _scatter_kernel(g_hbm, src_hbm, dst_hbm, out_hbm,
                    acc, sbuf, dbuf, rb0, rb1, sem0, sem1, six):
    cid = lax.axis_index("c")
    sid = lax.axis_index("s")
    wid = cid * NS + sid

    ix_s = pltpu.async_copy(src_hbm.at[pl.ds(wid * NCH, NCH)], sbuf, six)
    ix_d = pltpu.async_copy(dst_hbm.at[pl.ds(wid * NCH, NCH)], dbuf, six)

    def zero_rows(i, carry):
        def zrow(j, inner):
            rb0[i, pl.ds(j * 16, 16)] = jnp.zeros((16,), jnp.float32)
            return inner
        return lax.fori_loop(0, D // 16, zrow, carry)

    lax.fori_loop(0, CH, zero_rows, 0)

    for k in range(RPT // CH):
        pltpu.sync_copy(rb0, acc.at[pl.ds(sid * RPT + k * CH, CH)])
    rem = RPT - (RPT // CH) * CH
    if rem:
        pltpu.sync_copy(rb0.at[pl.ds(0, rem)],
                        acc.at[pl.ds(sid * RPT + (RPT // CH) * CH, rem)])
    ix_s.wait()
    ix_d.wait()
    plsc.subcore_barrier()

    pltpu.async_copy(g_hbm.at[sbuf.at[0]], rb0, sem0)

    def pair_body(p, carry):
        k0 = 2 * p
        k1 = k0 + 1
        k2 = k0 + 2
        pltpu.async_copy(g_hbm.at[sbuf.at[k1]], rb1, sem1)
        pltpu.make_async_copy(g_hbm.at[sbuf.at[k0]], rb0, sem0).wait()
        pltpu.sync_copy(rb0, acc.at[dbuf.at[k0]], add=True)
        pltpu.async_copy(g_hbm.at[sbuf.at[k2]], rb0, sem0)
        pltpu.make_async_copy(g_hbm.at[sbuf.at[k1]], rb1, sem1).wait()
        pltpu.sync_copy(rb1, acc.at[dbuf.at[k1]], add=True)
        return carry

    lax.fori_loop(0, (NCH - 1) // 2, pair_body, 0)
    pltpu.make_async_copy(g_hbm.at[sbuf.at[NCH - 1]], rb0, sem0).wait()
    pltpu.sync_copy(rb0, acc.at[dbuf.at[NCH - 1]], add=True)

    plsc.subcore_barrier()
    pltpu.sync_copy(acc.at[pl.ds(sid * RPT, RPT)],
                    out_hbm.at[cid, pl.ds(sid * RPT, RPT), :])


def _in_body(x_ref, wi_ref, bi_ref, w0_ref, dp_ref, h_ref, g_ref, dc_ref):
    cd = (((1,), (1,)), ((), ()))
    h = lax.dot_general(x_ref[...], wi_ref[...], cd,
                        preferred_element_type=jnp.float32) + bi_ref[...]
    h_ref[...] = h
    hw = lax.dot_general(h, w0_ref[...], cd,
                         preferred_element_type=jnp.float32)
    ones = jnp.ones((NW, 1), jnp.float32)
    s = lax.dot_general(dp_ref[...], ones, (((0,), (0,)), ((), ())),
                        preferred_element_type=jnp.float32)
    dinv = lax.rsqrt(1.0 + s)
    dc_ref[...] = dinv
    g_ref[...] = hw * dinv


def _mid_body(h_ref, g_ref, pa_ref, pb_ref, dinv_ref, cb_ref, lg_ref, lb_ref,
              w1_ref, h1_ref, g1_ref):
    dinv = dinv_ref[...]
    y = dinv * (pa_ref[0] + pb_ref[0] + g_ref[...]) + cb_ref[...]
    m = jnp.mean(y, axis=1, keepdims=True)
    c = y - m
    v = jnp.mean(c * c, axis=1, keepdims=True)
    yn = c * lax.rsqrt(v + 1e-5) * lg_ref[...] + lb_ref[...]
    h1 = h_ref[...] + jnp.maximum(yn, 0.0)
    h1_ref[...] = h1
    cd = (((1,), (1,)), ((), ()))
    g1_ref[...] = lax.dot_general(h1, w1_ref[...], cd,
                                  preferred_element_type=jnp.float32) * dinv


def _out_body(h_ref, g_ref, pa_ref, pb_ref, dinv_ref, cb_ref, lg_ref, lb_ref,
              ow_ref, ob_ref, o_ref):
    dinv = dinv_ref[...]
    y = dinv * (pa_ref[0] + pb_ref[0] + g_ref[...]) + cb_ref[...]
    m = jnp.mean(y, axis=1, keepdims=True)
    c = y - m
    v = jnp.mean(c * c, axis=1, keepdims=True)
    yn = c * lax.rsqrt(v + 1e-5) * lg_ref[...] + lb_ref[...]
    h2 = h_ref[...] + jnp.maximum(yn, 0.0)
    cd = (((1,), (1,)), ((), ()))
    o_ref[...] = lax.dot_general(h2, ow_ref[...], cd,
                                 preferred_element_type=jnp.float32) + ob_ref[...]


def _row_spec():
    return pl.BlockSpec((BR, D), lambda i: (i, 0))


def _w_spec():
    return pl.BlockSpec((D, D), lambda i: (0, 0))


def _v_spec():
    return pl.BlockSpec((1, D), lambda i: (0, 0))


def _dinv_spec():
    return pl.BlockSpec((BR, 1), lambda i: (i, 0))


def _p_spec(c):
    return pl.BlockSpec((1, BR, D), lambda i, c=c: (c, i, 0))


_f32 = jnp.float32


def kernel(x, edge_index, in_w, in_b, conv_w0, conv_b0, ln_g0, ln_b0,
           conv_w1, conv_b1, ln_g1, ln_b1, out_w, out_b):
    src2d = edge_index[0].reshape(E // CH, CH)
    dst2d = edge_index[1].reshape(E // CH, CH)
    dst_flat = edge_index[1]

    degp = _deg_kernel(dst_flat)

    h, g0, dinv_col = pl.pallas_call(
        _in_body,
        grid=(GRID1,),
        in_specs=[pl.BlockSpec((BR1, D), lambda i: (i, 0)),
                  pl.BlockSpec((D, D), lambda i: (0, 0)),
                  pl.BlockSpec((1, D), lambda i: (0, 0)),
                  pl.BlockSpec((D, D), lambda i: (0, 0)),
                  pl.BlockSpec((NW, BR1), lambda i: (0, i))],
        out_specs=[pl.BlockSpec((BR1, D), lambda i: (i, 0)),
                   pl.BlockSpec((BR1, D), lambda i: (i, 0)),
                   pl.BlockSpec((BR1, 1), lambda i: (i, 0))],
        out_shape=[jax.ShapeDtypeStruct((N, D), _f32),
                   jax.ShapeDtypeStruct((N, D), _f32),
                   jax.ShapeDtypeStruct((N, 1), _f32)],
    )(x, in_w, in_b.reshape(1, D), conv_w0, degp)

    p0 = _scatter_kernel(g0, src2d, dst2d)

    h1, g1 = pl.pallas_call(
        _mid_body,
        grid=(GRID,),
        in_specs=[_row_spec(), _row_spec(), _p_spec(0), _p_spec(1),
                  _dinv_spec(), _v_spec(), _v_spec(), _v_spec(), _w_spec()],
        out_specs=[_row_spec(), _row_spec()],
        out_shape=[jax.ShapeDtypeStruct((N, D), _f32),
                   jax.ShapeDtypeStruct((N, D), _f32)],
    )(h, g0, p0, p0, dinv_col, conv_b0.reshape(1, D), ln_g0.reshape(1, D),
      ln_b0.reshape(1, D), conv_w1)

    p1 = _scatter_kernel(g1, src2d, dst2d)

    out = pl.pallas_call(
        _out_body,
        grid=(GRID,),
        in_specs=[_row_spec(), _row_spec(), _p_spec(0), _p_spec(1),
                  _dinv_spec(), _v_spec(), _v_spec(), _v_spec(), _w_spec(),
                  _v_spec()],
        out_specs=_row_spec(),
        out_shape=jax.ShapeDtypeStruct((N, D), _f32),
    )(h1, g1, p1, p1, dinv_col, conv_b1.reshape(1, D), ln_g1.reshape(1, D),
      ln_b1.reshape(1, D), out_w, out_b.reshape(1, D))

    return out


# speedup vs baseline: 4.5625x; 1.0362x over previous
import functools

import jax
import jax.numpy as jnp
from jax import lax
from jax.experimental import pallas as pl
from jax.experimental.pallas import tpu as pltpu
from jax.experimental.pallas import tpu_sc as plsc

N = 10000
E = 320000
D = 128

NC = 2
NS = 16
NW = NC * NS
EPW = E // NW
CH = 100
NCH = EPW // CH
RPT = N // NS
ZR = 125
BR = 1000
GRID = N // BR
BR1 = 1024
GRID1 = 10

DEGP = 10240

_mesh = plsc.VectorSubcoreMesh(core_axis_name="c", subcore_axis_name="s")


@functools.partial(
    pl.kernel,
    out_type=jax.ShapeDtypeStruct((NW, DEGP), jnp.float32),
    mesh=_mesh,
    compiler_params=pltpu.CompilerParams(needs_layout_passes=False, use_tc_tiling_on_sc=False),
    scratch_types=[
        pltpu.VMEM((DEGP,), jnp.float32),
        pltpu.VMEM((EPW,), jnp.int32),
    ],
)
def _deg_kernel(dst_hbm, out_hbm, dv, dbuf):
    cid = lax.axis_index("c")
    sid = lax.axis_index("s")
    wid = cid * NS + sid

    def zero_body(i, carry):
        dv[pl.ds(i * 16, 16)] = jnp.zeros((16,), jnp.float32)
        return carry

    lax.fori_loop(0, DEGP // 16, zero_body, 0)

    base = pl.multiple_of(wid * EPW, 8)
    pltpu.sync_copy(dst_hbm.at[pl.ds(base, EPW)], dbuf)

    ones = jnp.ones((16,), jnp.float32)

    def acc_body(j, carry):
        idx = dbuf[pl.ds(j * 16, 16)]
        plsc.addupdate_scatter(dv, [idx], ones)
        return carry

    lax.fori_loop(0, EPW // 16, acc_body, 0)
    pltpu.sync_copy(dv, out_hbm.at[wid])


@functools.partial(
    pl.kernel,
    out_type=jax.ShapeDtypeStruct((NC, N, D), jnp.float32),
    mesh=_mesh,
    compiler_params=pltpu.CompilerParams(needs_layout_passes=False, use_tc_tiling_on_sc=False),
    scratch_types=[
        pltpu.VMEM_SHARED((N, D), jnp.float32),
        pltpu.VMEM((NCH, CH), jnp.int32),
        pltpu.VMEM((NCH, CH), jnp.int32),
        pltpu.VMEM((CH, D), jnp.float32),
        pltpu.VMEM((CH, D), jnp.float32),
        pltpu.SemaphoreType.DMA,
        pltpu.SemaphoreType.DMA,
        pltpu.SemaphoreType.DMA,
    ],
)
def _scatter_kernel(g_hbm, src_hbm, dst_hbm, out_hbm,
                    acc, sbuf, dbuf, rb0, rb1, sem0, sem1, six):
    cid = lax.axis_index("c")
    sid = lax.axis_index("s")
    wid = cid * NS + sid

    ix_s = pltpu.async_copy(src_hbm.at[pl.ds(wid * NCH, NCH)], sbuf, six)
    ix_d = pltpu.async_copy(dst_hbm.at[pl.ds(wid * NCH, NCH)], dbuf, six)

    def zero_rows(i, carry):
        def zrow(j, inner):
            rb0[i, pl.ds(j * 16, 16)] = jnp.zeros((16,), jnp.float32)
            return inner
        return lax.fori_loop(0, D // 16, zrow, carry)

    lax.fori_loop(0, CH, zero_rows, 0)

    for k in range(RPT // CH):
        pltpu.sync_copy(rb0, acc.at[pl.ds(sid * RPT + k * CH, CH)])
    rem = RPT - (RPT // CH) * CH
    if rem:
        pltpu.sync_copy(rb0.at[pl.ds(0, rem)],
                        acc.at[pl.ds(sid * RPT + (RPT // CH) * CH, rem)])
    ix_s.wait()
    ix_d.wait()
    plsc.subcore_barrier()

    pltpu.async_copy(g_hbm.at[sbuf.at[0]], rb0, sem0)

    def pair_body(p, carry):
        k0 = 2 * p
        k1 = k0 + 1
        k2 = k0 + 2
        pltpu.async_copy(g_hbm.at[sbuf.at[k1]], rb1, sem1)
        pltpu.make_async_copy(g_hbm.at[sbuf.at[k0]], rb0, sem0).wait()
        pltpu.sync_copy(rb0, acc.at[dbuf.at[k0]], add=True)
        pltpu.async_copy(g_hbm.at[sbuf.at[k2]], rb0, sem0)
        pltpu.make_async_copy(g_hbm.at[sbuf.at[k1]], rb1, sem1).wait()
        pltpu.sync_copy(rb1, acc.at[dbuf.at[k1]], add=True)
        return carry

    lax.fori_loop(0, (NCH - 1) // 2, pair_body, 0)
    if NCH % 2 == 0:
        pltpu.async_copy(g_hbm.at[sbuf.at[NCH - 1]], rb1, sem1)
        pltpu.make_async_copy(g_hbm.at[sbuf.at[NCH - 2]], rb0, sem0).wait()
        pltpu.sync_copy(rb0, acc.at[dbuf.at[NCH - 2]], add=True)
        pltpu.make_async_copy(g_hbm.at[sbuf.at[NCH - 1]], rb1, sem1).wait()
        pltpu.sync_copy(rb1, acc.at[dbuf.at[NCH - 1]], add=True)
    else:
        pltpu.make_async_copy(g_hbm.at[sbuf.at[NCH - 1]], rb0, sem0).wait()
        pltpu.sync_copy(rb0, acc.at[dbuf.at[NCH - 1]], add=True)

    plsc.subcore_barrier()
    pltpu.sync_copy(acc.at[pl.ds(sid * RPT, RPT)],
                    out_hbm.at[cid, pl.ds(sid * RPT, RPT), :])


def _in_body(x_ref, wi_ref, bi_ref, w0_ref, dp_ref, h_ref, g_ref, dc_ref):
    cd = (((1,), (1,)), ((), ()))
    h = lax.dot_general(x_ref[...], wi_ref[...], cd,
                        preferred_element_type=jnp.float32) + bi_ref[...]
    h_ref[...] = h
    hw = lax.dot_general(h, w0_ref[...], cd,
                         preferred_element_type=jnp.float32)
    ones = jnp.ones((NW, 1), jnp.float32)
    s = lax.dot_general(dp_ref[...], ones, (((0,), (0,)), ((), ())),
                        preferred_element_type=jnp.float32)
    dinv = lax.rsqrt(1.0 + s)
    dc_ref[...] = dinv
    g_ref[...] = hw * dinv


def _mid_body(h_ref, g_ref, pa_ref, pb_ref, dinv_ref, cb_ref, lg_ref, lb_ref,
              w1_ref, h1_ref, g1_ref):
    dinv = dinv_ref[...]
    y = dinv * (pa_ref[0] + pb_ref[0] + g_ref[...]) + cb_ref[...]
    m = jnp.mean(y, axis=1, keepdims=True)
    c = y - m
    v = jnp.mean(c * c, axis=1, keepdims=True)
    yn = c * lax.rsqrt(v + 1e-5) * lg_ref[...] + lb_ref[...]
    h1 = h_ref[...] + jnp.maximum(yn, 0.0)
    h1_ref[...] = h1
    cd = (((1,), (1,)), ((), ()))
    g1_ref[...] = lax.dot_general(h1, w1_ref[...], cd,
                                  preferred_element_type=jnp.float32) * dinv


def _out_body(h_ref, g_ref, pa_ref, pb_ref, dinv_ref, cb_ref, lg_ref, lb_ref,
              ow_ref, ob_ref, o_ref):
    dinv = dinv_ref[...]
    y = dinv * (pa_ref[0] + pb_ref[0] + g_ref[...]) + cb_ref[...]
    m = jnp.mean(y, axis=1, keepdims=True)
    c = y - m
    v = jnp.mean(c * c, axis=1, keepdims=True)
    yn = c * lax.rsqrt(v + 1e-5) * lg_ref[...] + lb_ref[...]
    h2 = h_ref[...] + jnp.maximum(yn, 0.0)
    cd = (((1,), (1,)), ((), ()))
    o_ref[...] = lax.dot_general(h2, ow_ref[...], cd,
                                 preferred_element_type=jnp.float32) + ob_ref[...]


def _row_spec():
    return pl.BlockSpec((BR, D), lambda i: (i, 0))


def _w_spec():
    return pl.BlockSpec((D, D), lambda i: (0, 0))


def _v_spec():
    return pl.BlockSpec((1, D), lambda i: (0, 0))


def _dinv_spec():
    return pl.BlockSpec((BR, 1), lambda i: (i, 0))


def _p_spec(c):
    return pl.BlockSpec((1, BR, D), lambda i, c=c: (c, i, 0))


_f32 = jnp.float32


def kernel(x, edge_index, in_w, in_b, conv_w0, conv_b0, ln_g0, ln_b0,
           conv_w1, conv_b1, ln_g1, ln_b1, out_w, out_b):
    src2d = edge_index[0].reshape(E // CH, CH)
    dst2d = edge_index[1].reshape(E // CH, CH)
    dst_flat = edge_index[1]

    degp = _deg_kernel(dst_flat)

    h, g0, dinv_col = pl.pallas_call(
        _in_body,
        grid=(GRID1,),
        in_specs=[pl.BlockSpec((BR1, D), lambda i: (i, 0)),
                  pl.BlockSpec((D, D), lambda i: (0, 0)),
                  pl.BlockSpec((1, D), lambda i: (0, 0)),
                  pl.BlockSpec((D, D), lambda i: (0, 0)),
                  pl.BlockSpec((NW, BR1), lambda i: (0, i))],
        out_specs=[pl.BlockSpec((BR1, D), lambda i: (i, 0)),
                   pl.BlockSpec((BR1, D), lambda i: (i, 0)),
                   pl.BlockSpec((BR1, 1), lambda i: (i, 0))],
        out_shape=[jax.ShapeDtypeStruct((N, D), _f32),
                   jax.ShapeDtypeStruct((N, D), _f32),
                   jax.ShapeDtypeStruct((N, 1), _f32)],
    )(x, in_w, in_b.reshape(1, D), conv_w0, degp)

    p0 = _scatter_kernel(g0, src2d, dst2d)

    h1, g1 = pl.pallas_call(
        _mid_body,
        grid=(GRID,),
        in_specs=[_row_spec(), _row_spec(), _p_spec(0), _p_spec(1),
                  _dinv_spec(), _v_spec(), _v_spec(), _v_spec(), _w_spec()],
        out_specs=[_row_spec(), _row_spec()],
        out_shape=[jax.ShapeDtypeStruct((N, D), _f32),
                   jax.ShapeDtypeStruct((N, D), _f32)],
    )(h, g0, p0, p0, dinv_col, conv_b0.reshape(1, D), ln_g0.reshape(1, D),
      ln_b0.reshape(1, D), conv_w1)

    p1 = _scatter_kernel(g1, src2d, dst2d)

    out = pl.pallas_call(
        _out_body,
        grid=(GRID,),
        in_specs=[_row_spec(), _row_spec(), _p_spec(0), _p_spec(1),
                  _dinv_spec(), _v_spec(), _v_spec(), _v_spec(), _w_spec(),
                  _v_spec()],
        out_specs=_row_spec(),
        out_shape=jax.ShapeDtypeStruct((N, D), _f32),
    )(h1, g1, p1, p1, dinv_col, conv_b1.reshape(1, D), ln_g1.reshape(1, D),
      ln_b1.reshape(1, D), out_w, out_b.reshape(1, D))

    return out
